# baseline (speedup 1.0000x reference)
import functools

import jax
import jax.numpy as jnp
from jax import lax
from jax.experimental import pallas as pl
from jax.experimental.pallas import tpu as pltpu
from jax.experimental.pallas import tpu_sc as plsc

T = 2048
D = 1024
F = 3584
E = 8
K = 2
P = T * K
TM = 128
PPAD = P + E * TM
NB = PPAD // TM
FN = 512
DN = 512


def _router_body(x_ref, gw_ref, logits_ref, rw_ref, sel_ref):
    x = x_ref[...]
    logits = jnp.dot(x, gw_ref[...], preferred_element_type=jnp.float32)
    logits_ref[...] = logits
    iota = lax.broadcasted_iota(jnp.int32, logits.shape, 1)
    m1 = jnp.max(logits, axis=1, keepdims=True)
    a1 = jnp.min(jnp.where(logits == m1, iota, E), axis=1, keepdims=True)
    rest = jnp.where(iota == a1, -jnp.inf, logits)
    m2 = jnp.max(rest, axis=1, keepdims=True)
    a2 = jnp.min(jnp.where(rest == m2, iota, E), axis=1, keepdims=True)
    e2 = jnp.exp(m2 - m1)
    denom = 1.0 + e2
    rw_ref[...] = jnp.concatenate([1.0 / denom, e2 / denom], axis=1)
    sel_ref[...] = jnp.concatenate([a1, a2], axis=1)


def _router(x, gate_w):
    bt = 256
    return pl.pallas_call(
        _router_body,
        grid=(T // bt,),
        in_specs=[
            pl.BlockSpec((bt, D), lambda i: (i, 0)),
            pl.BlockSpec((D, E), lambda i: (0, 0)),
        ],
        out_specs=[
            pl.BlockSpec((bt, E), lambda i: (i, 0)),
            pl.BlockSpec((bt, K), lambda i: (i, 0)),
            pl.BlockSpec((bt, K), lambda i: (i, 0)),
        ],
        out_shape=[
            jax.ShapeDtypeStruct((T, E), jnp.float32),
            jax.ShapeDtypeStruct((T, K), jnp.float32),
            jax.ShapeDtypeStruct((T, K), jnp.int32),
        ],
    )(x, gate_w)


def _prep_body(sel_ref, dest_ref, be_ref):
    onehot = (sel_ref[...] == lax.broadcasted_iota(jnp.int32, (P, E), 1)
              ).astype(jnp.float32)
    ch = 512
    tri = (lax.broadcasted_iota(jnp.int32, (ch, ch), 0)
           >= lax.broadcasted_iota(jnp.int32, (ch, ch), 1)).astype(jnp.float32)
    running = jnp.zeros((1, E), jnp.float32)
    ranks = []
    for i in range(P // ch):
        blk = onehot[i * ch:(i + 1) * ch, :]
        ranks.append(jnp.dot(tri, blk, preferred_element_type=jnp.float32)
                     + running)
        running = running + jnp.sum(blk, axis=0, keepdims=True)
    rank = jnp.concatenate(ranks, axis=0)
    counts = running
    padded = jnp.floor((counts + (TM - 1)) / TM) * TM
    triu8 = (lax.broadcasted_iota(jnp.int32, (E, E), 0)
             < lax.broadcasted_iota(jnp.int32, (E, E), 1)).astype(jnp.float32)
    offsets = jnp.dot(padded, triu8, preferred_element_type=jnp.float32)
    dest_f = jnp.sum(onehot * (offsets + rank - 1.0), axis=1, keepdims=True)
    dest_ref[...] = dest_f.astype(jnp.int32)

    rb = (lax.broadcasted_iota(jnp.int32, (NB, E), 0) * TM).astype(jnp.float32)
    be_ref[...] = (jnp.sum((rb >= offsets).astype(jnp.float32),
                           axis=1, keepdims=True) - 1.0).astype(jnp.int32)


def _prep(sel_col):
    return pl.pallas_call(
        _prep_body,
        in_specs=[pl.BlockSpec((P, 1), lambda: (0, 0))],
        out_specs=[
            pl.BlockSpec((P, 1), lambda: (0, 0)),
            pl.BlockSpec((NB, 1), lambda: (0, 0)),
        ],
        out_shape=[
            jax.ShapeDtypeStruct((P, 1), jnp.int32),
            jax.ShapeDtypeStruct((NB, 1), jnp.int32),
        ],
    )(sel_col)


def _sc_dispatch(x, idx_a, idx_b):
    info = plsc.get_sparse_core_info()
    nw = info.num_cores * info.num_subcores
    t_per_w = T // nw
    mesh = plsc.VectorSubcoreMesh(core_axis_name="c", subcore_axis_name="s")

    @functools.partial(
        pl.kernel, mesh=mesh,
        out_type=jax.ShapeDtypeStruct((PPAD, D), jnp.float32),
        scratch_types=[
            pltpu.VMEM((t_per_w,), jnp.int32),
            pltpu.VMEM((t_per_w,), jnp.int32),
            pltpu.VMEM((t_per_w, D), jnp.float32),
            pltpu.SemaphoreType.DMA,
            pltpu.SemaphoreType.DMA,
        ],
    )
    def k(x_hbm, ia_hbm, ib_hbm, out_hbm, ia_v, ib_v, rows_v, sem_a, sem_b):
        wid = lax.axis_index("s") * info.num_cores + lax.axis_index("c")
        base = wid * t_per_w
        pltpu.sync_copy(ia_hbm.at[pl.ds(base, t_per_w)], ia_v)
        pltpu.sync_copy(ib_hbm.at[pl.ds(base, t_per_w)], ib_v)
        pltpu.sync_copy(x_hbm.at[pl.ds(base, t_per_w)], rows_v)
        ca = pltpu.async_copy(rows_v, out_hbm.at[ia_v], sem_a)
        cb = pltpu.async_copy(rows_v, out_hbm.at[ib_v], sem_b)
        ca.wait()
        cb.wait()

    return k(x, idx_a, idx_b)


def _sc_collect(table, idx):
    info = plsc.get_sparse_core_info()
    nw = info.num_cores * info.num_subcores
    b_per_w = P // nw
    chunk = 64
    mesh = plsc.VectorSubcoreMesh(core_axis_name="c", subcore_axis_name="s")

    @functools.partial(
        pl.kernel, mesh=mesh,
        out_type=jax.ShapeDtypeStruct((P, D), jnp.float32),
        scratch_types=[
            pltpu.VMEM((chunk,), jnp.int32),
            pltpu.VMEM((chunk, D), jnp.float32),
            pltpu.SemaphoreType.DMA,
        ],
    )
    def k(table_hbm, idx_hbm, out_hbm, idx_v, rows_v, sem):
        wid = lax.axis_index("s") * info.num_cores + lax.axis_index("c")
        base = wid * b_per_w
        for c in range(b_per_w // chunk):
            off = base + c * chunk
            pltpu.sync_copy(idx_hbm.at[pl.ds(off, chunk)], idx_v)
            pltpu.async_copy(table_hbm.at[idx_v], rows_v, sem).wait()
            pltpu.sync_copy(rows_v, out_hbm.at[pl.ds(off, chunk)])

    return k(table, idx)


def _stage_a_body(be_ref, xs_ref, w1_ref, w3_ref, h_ref):
    xs = xs_ref[...]
    a = jnp.dot(xs, w1_ref[0], preferred_element_type=jnp.float32,
                precision=lax.Precision.DEFAULT)
    b = jnp.dot(xs, w3_ref[0], preferred_element_type=jnp.float32,
                precision=lax.Precision.DEFAULT)
    h_ref[...] = a * jax.nn.sigmoid(a) * b


def _stage_a(xs, w1, w3, be):
    grid = (F // FN, NB)
    return pl.pallas_call(
        _stage_a_body,
        grid_spec=pltpu.PrefetchScalarGridSpec(
            num_scalar_prefetch=1,
            grid=grid,
            in_specs=[
                pl.BlockSpec((TM, D), lambda fb, rb, be: (rb, 0)),
                pl.BlockSpec((1, D, FN), lambda fb, rb, be: (be[rb], 0, fb)),
                pl.BlockSpec((1, D, FN), lambda fb, rb, be: (be[rb], 0, fb)),
            ],
            out_specs=pl.BlockSpec((TM, FN), lambda fb, rb, be: (rb, fb)),
        ),
        out_shape=jax.ShapeDtypeStruct((PPAD, F), jnp.float32),
    )(be, xs, w1, w3)


def _stage_b_body(be_ref, h_ref, w2_ref, op_ref):
    op_ref[...] = jnp.dot(h_ref[...], w2_ref[0],
                          preferred_element_type=jnp.float32,
                          precision=lax.Precision.DEFAULT)


def _stage_b(h, w2, be):
    grid = (D // DN, NB)
    return pl.pallas_call(
        _stage_b_body,
        grid_spec=pltpu.PrefetchScalarGridSpec(
            num_scalar_prefetch=1,
            grid=grid,
            in_specs=[
                pl.BlockSpec((TM, F), lambda db, rb, be: (rb, 0)),
                pl.BlockSpec((1, F, DN), lambda db, rb, be: (be[rb], 0, db)),
            ],
            out_specs=pl.BlockSpec((TM, DN), lambda db, rb, be: (rb, db)),
        ),
        out_shape=jax.ShapeDtypeStruct((PPAD, D), jnp.float32),
    )(be, h, w2)


def _combine_body(g_ref, rw_ref, out_ref):
    g = g_ref[...]
    rw = rw_ref[...]
    out_ref[...] = (g[:, :D] * rw[:, 0:1] + g[:, D:] * rw[:, 1:2])[None]


def _combine(g2, rw):
    bt = 256
    return pl.pallas_call(
        _combine_body,
        grid=(T // bt,),
        in_specs=[
            pl.BlockSpec((bt, K * D), lambda i: (i, 0)),
            pl.BlockSpec((bt, K), lambda i: (i, 0)),
        ],
        out_specs=pl.BlockSpec((1, bt, D), lambda i: (0, i, 0)),
        out_shape=jax.ShapeDtypeStruct((1, T, D), jnp.float32),
    )(g2, rw)


def kernel(hidden_states, gate_w, w1, w2, w3):
    x = hidden_states.reshape(T, D)
    router_logits, rw, sel = _router(x, gate_w)

    dest, be2 = _prep(sel.reshape(P, 1))
    be = be2.reshape(NB)
    dest2 = dest.reshape(T, K)
    idx_a = dest2[:, 0]
    idx_b = dest2[:, 1]
    pos = dest.reshape(P)

    xs = _sc_dispatch(x, idx_a, idx_b)
    h = _stage_a(xs, w1, w3, be)
    op = _stage_b(h, w2, be)
    g = _sc_collect(op, pos)
    out = _combine(g.reshape(T, K * D), rw)
    return out, router_logits


# speedup vs baseline: 1.2421x; 1.2421x over previous
import functools

import jax
import jax.numpy as jnp
from jax import lax
from jax.experimental import pallas as pl
from jax.experimental.pallas import tpu as pltpu
from jax.experimental.pallas import tpu_sc as plsc

T = 2048
D = 1024
F = 3584
E = 8
K = 2
P = T * K
TM = 256
PPAD = P + E * TM
NB = PPAD // TM
FN = 896
DN = 512


def _router_body(x_ref, gw_ref, logits_ref, rw_ref, sel_ref):
    x = x_ref[...]
    logits = jnp.dot(x, gw_ref[...], preferred_element_type=jnp.float32)
    logits_ref[...] = logits
    iota = lax.broadcasted_iota(jnp.int32, logits.shape, 1)
    m1 = jnp.max(logits, axis=1, keepdims=True)
    a1 = jnp.min(jnp.where(logits == m1, iota, E), axis=1, keepdims=True)
    rest = jnp.where(iota == a1, -jnp.inf, logits)
    m2 = jnp.max(rest, axis=1, keepdims=True)
    a2 = jnp.min(jnp.where(rest == m2, iota, E), axis=1, keepdims=True)
    e2 = jnp.exp(m2 - m1)
    denom = 1.0 + e2
    rw_ref[...] = jnp.concatenate([1.0 / denom, e2 / denom], axis=1)
    sel_ref[...] = jnp.concatenate([a1, a2], axis=1)


def _router(x, gate_w):
    bt = 256
    return pl.pallas_call(
        _router_body,
        grid=(T // bt,),
        in_specs=[
            pl.BlockSpec((bt, D), lambda i: (i, 0)),
            pl.BlockSpec((D, E), lambda i: (0, 0)),
        ],
        out_specs=[
            pl.BlockSpec((bt, E), lambda i: (i, 0)),
            pl.BlockSpec((bt, K), lambda i: (i, 0)),
            pl.BlockSpec((bt, K), lambda i: (i, 0)),
        ],
        out_shape=[
            jax.ShapeDtypeStruct((T, E), jnp.float32),
            jax.ShapeDtypeStruct((T, K), jnp.float32),
            jax.ShapeDtypeStruct((T, K), jnp.int32),
        ],
    )(x, gate_w)


def _prep_body(sel_ref, dest_ref, be_ref):
    onehot = (sel_ref[...] == lax.broadcasted_iota(jnp.int32, (P, E), 1)
              ).astype(jnp.float32)
    ch = 512
    tri = (lax.broadcasted_iota(jnp.int32, (ch, ch), 0)
           >= lax.broadcasted_iota(jnp.int32, (ch, ch), 1)).astype(jnp.float32)
    running = jnp.zeros((1, E), jnp.float32)
    ranks = []
    for i in range(P // ch):
        blk = onehot[i * ch:(i + 1) * ch, :]
        ranks.append(jnp.dot(tri, blk, preferred_element_type=jnp.float32)
                     + running)
        running = running + jnp.sum(blk, axis=0, keepdims=True)
    rank = jnp.concatenate(ranks, axis=0)
    counts = running
    padded = jnp.floor((counts + (TM - 1)) / TM) * TM
    triu8 = (lax.broadcasted_iota(jnp.int32, (E, E), 0)
             < lax.broadcasted_iota(jnp.int32, (E, E), 1)).astype(jnp.float32)
    offsets = jnp.dot(padded, triu8, preferred_element_type=jnp.float32)
    dest_f = jnp.sum(onehot * (offsets + rank - 1.0), axis=1, keepdims=True)
    dest_ref[...] = dest_f.astype(jnp.int32)

    rb = (lax.broadcasted_iota(jnp.int32, (NB, E), 0) * TM).astype(jnp.float32)
    be_ref[...] = (jnp.sum((rb >= offsets).astype(jnp.float32),
                           axis=1, keepdims=True) - 1.0).astype(jnp.int32)


def _prep(sel_col):
    return pl.pallas_call(
        _prep_body,
        in_specs=[pl.BlockSpec((P, 1), lambda: (0, 0))],
        out_specs=[
            pl.BlockSpec((P, 1), lambda: (0, 0)),
            pl.BlockSpec((NB, 1), lambda: (0, 0)),
        ],
        out_shape=[
            jax.ShapeDtypeStruct((P, 1), jnp.int32),
            jax.ShapeDtypeStruct((NB, 1), jnp.int32),
        ],
    )(sel_col)


def _sc_dispatch(x, idx_a, idx_b):
    info = plsc.get_sparse_core_info()
    nw = info.num_cores * info.num_subcores
    t_per_w = T // nw
    mesh = plsc.VectorSubcoreMesh(core_axis_name="c", subcore_axis_name="s")

    @functools.partial(
        pl.kernel, mesh=mesh,
        out_type=jax.ShapeDtypeStruct((PPAD, D), jnp.float32),
        scratch_types=[
            pltpu.VMEM((t_per_w,), jnp.int32),
            pltpu.VMEM((t_per_w,), jnp.int32),
            pltpu.VMEM((t_per_w, D), jnp.float32),
            pltpu.SemaphoreType.DMA,
            pltpu.SemaphoreType.DMA,
        ],
    )
    def k(x_hbm, ia_hbm, ib_hbm, out_hbm, ia_v, ib_v, rows_v, sem_a, sem_b):
        wid = lax.axis_index("s") * info.num_cores + lax.axis_index("c")
        base = wid * t_per_w
        pltpu.sync_copy(ia_hbm.at[pl.ds(base, t_per_w)], ia_v)
        pltpu.sync_copy(ib_hbm.at[pl.ds(base, t_per_w)], ib_v)
        pltpu.sync_copy(x_hbm.at[pl.ds(base, t_per_w)], rows_v)
        ca = pltpu.async_copy(rows_v, out_hbm.at[ia_v], sem_a)
        cb = pltpu.async_copy(rows_v, out_hbm.at[ib_v], sem_b)
        ca.wait()
        cb.wait()

    return k(x, idx_a, idx_b)


def _sc_collect(table, idx):
    info = plsc.get_sparse_core_info()
    nw = info.num_cores * info.num_subcores
    b_per_w = P // nw
    chunk = 64
    mesh = plsc.VectorSubcoreMesh(core_axis_name="c", subcore_axis_name="s")

    @functools.partial(
        pl.kernel, mesh=mesh,
        out_type=jax.ShapeDtypeStruct((P, D), jnp.float32),
        scratch_types=[
            pltpu.VMEM((chunk,), jnp.int32),
            pltpu.VMEM((chunk, D), jnp.float32),
            pltpu.SemaphoreType.DMA,
        ],
    )
    def k(table_hbm, idx_hbm, out_hbm, idx_v, rows_v, sem):
        wid = lax.axis_index("s") * info.num_cores + lax.axis_index("c")
        base = wid * b_per_w
        for c in range(b_per_w // chunk):
            off = base + c * chunk
            pltpu.sync_copy(idx_hbm.at[pl.ds(off, chunk)], idx_v)
            pltpu.async_copy(table_hbm.at[idx_v], rows_v, sem).wait()
            pltpu.sync_copy(rows_v, out_hbm.at[pl.ds(off, chunk)])

    return k(table, idx)


def _stage_a_body(be_ref, xs_ref, w1_ref, w3_ref, h_ref):
    xs = xs_ref[...]
    a = jnp.dot(xs, w1_ref[0], preferred_element_type=jnp.float32,
                precision=lax.Precision.DEFAULT)
    b = jnp.dot(xs, w3_ref[0], preferred_element_type=jnp.float32,
                precision=lax.Precision.DEFAULT)
    h_ref[...] = a * jax.nn.sigmoid(a) * b


def _stage_a(xs, w1, w3, be):
    grid = (F // FN, NB)
    return pl.pallas_call(
        _stage_a_body,
        grid_spec=pltpu.PrefetchScalarGridSpec(
            num_scalar_prefetch=1,
            grid=grid,
            in_specs=[
                pl.BlockSpec((TM, D), lambda fb, rb, be: (rb, 0)),
                pl.BlockSpec((1, D, FN), lambda fb, rb, be: (be[rb], 0, fb)),
                pl.BlockSpec((1, D, FN), lambda fb, rb, be: (be[rb], 0, fb)),
            ],
            out_specs=pl.BlockSpec((TM, FN), lambda fb, rb, be: (rb, fb)),
        ),
        out_shape=jax.ShapeDtypeStruct((PPAD, F), jnp.float32),
    )(be, xs, w1, w3)


def _stage_b_body(be_ref, h_ref, w2_ref, op_ref):
    op_ref[...] = jnp.dot(h_ref[...], w2_ref[0],
                          preferred_element_type=jnp.float32,
                          precision=lax.Precision.DEFAULT)


def _stage_b(h, w2, be):
    grid = (D // DN, NB)
    return pl.pallas_call(
        _stage_b_body,
        grid_spec=pltpu.PrefetchScalarGridSpec(
            num_scalar_prefetch=1,
            grid=grid,
            in_specs=[
                pl.BlockSpec((TM, F), lambda db, rb, be: (rb, 0)),
                pl.BlockSpec((1, F, DN), lambda db, rb, be: (be[rb], 0, db)),
            ],
            out_specs=pl.BlockSpec((TM, DN), lambda db, rb, be: (rb, db)),
        ),
        out_shape=jax.ShapeDtypeStruct((PPAD, D), jnp.float32),
    )(be, h, w2)


def _combine_body(g_ref, rw_ref, out_ref):
    g = g_ref[...]
    rw = rw_ref[...]
    out_ref[...] = (g[:, :D] * rw[:, 0:1] + g[:, D:] * rw[:, 1:2])[None]


def _combine(g2, rw):
    bt = 256
    return pl.pallas_call(
        _combine_body,
        grid=(T // bt,),
        in_specs=[
            pl.BlockSpec((bt, K * D), lambda i: (i, 0)),
            pl.BlockSpec((bt, K), lambda i: (i, 0)),
        ],
        out_specs=pl.BlockSpec((1, bt, D), lambda i: (0, i, 0)),
        out_shape=jax.ShapeDtypeStruct((1, T, D), jnp.float32),
    )(g2, rw)


def kernel(hidden_states, gate_w, w1, w2, w3):
    x = hidden_states.reshape(T, D)
    router_logits, rw, sel = _router(x, gate_w)

    dest, be2 = _prep(sel.reshape(P, 1))
    be = be2.reshape(NB)
    dest2 = dest.reshape(T, K)
    idx_a = dest2[:, 0]
    idx_b = dest2[:, 1]
    pos = dest.reshape(P)

    xs = _sc_dispatch(x, idx_a, idx_b)
    h = _stage_a(xs, w1, w3, be)
    op = _stage_b(h, w2, be)
    g = _sc_collect(op, pos)
    out = _combine(g.reshape(T, K * D), rw)
    return out, router_logits


# speedup vs baseline: 1.4676x; 1.1816x over previous
import functools

import jax
import jax.numpy as jnp
from jax import lax
from jax.experimental import pallas as pl
from jax.experimental.pallas import tpu as pltpu
from jax.experimental.pallas import tpu_sc as plsc

T = 2048
D = 1024
F = 3584
E = 8
K = 2
P = T * K
TM = 256
PPAD = P + E * TM
NB = PPAD // TM
FN = 1792
DN = 1024


def _router_body(x_ref, gw_ref, logits_ref, rw_ref, sel_ref):
    x = x_ref[...]
    logits = jnp.dot(x, gw_ref[...], preferred_element_type=jnp.float32)
    logits_ref[...] = logits
    iota = lax.broadcasted_iota(jnp.int32, logits.shape, 1)
    m1 = jnp.max(logits, axis=1, keepdims=True)
    a1 = jnp.min(jnp.where(logits == m1, iota, E), axis=1, keepdims=True)
    rest = jnp.where(iota == a1, -jnp.inf, logits)
    m2 = jnp.max(rest, axis=1, keepdims=True)
    a2 = jnp.min(jnp.where(rest == m2, iota, E), axis=1, keepdims=True)
    e2 = jnp.exp(m2 - m1)
    denom = 1.0 + e2
    rw_ref[...] = jnp.concatenate([1.0 / denom, e2 / denom], axis=1)
    sel_ref[...] = jnp.concatenate([a1, a2], axis=1)


def _router(x, gate_w):
    bt = 256
    return pl.pallas_call(
        _router_body,
        grid=(T // bt,),
        in_specs=[
            pl.BlockSpec((bt, D), lambda i: (i, 0)),
            pl.BlockSpec((D, E), lambda i: (0, 0)),
        ],
        out_specs=[
            pl.BlockSpec((bt, E), lambda i: (i, 0)),
            pl.BlockSpec((bt, K), lambda i: (i, 0)),
            pl.BlockSpec((bt, K), lambda i: (i, 0)),
        ],
        out_shape=[
            jax.ShapeDtypeStruct((T, E), jnp.float32),
            jax.ShapeDtypeStruct((T, K), jnp.float32),
            jax.ShapeDtypeStruct((T, K), jnp.int32),
        ],
    )(x, gate_w)


def _prep_body(sel_ref, dest_ref, be_ref):
    onehot = (sel_ref[...] == lax.broadcasted_iota(jnp.int32, (P, E), 1)
              ).astype(jnp.float32)
    ch = 512
    tri = (lax.broadcasted_iota(jnp.int32, (ch, ch), 0)
           >= lax.broadcasted_iota(jnp.int32, (ch, ch), 1)).astype(jnp.float32)
    running = jnp.zeros((1, E), jnp.float32)
    ranks = []
    for i in range(P // ch):
        blk = onehot[i * ch:(i + 1) * ch, :]
        ranks.append(jnp.dot(tri, blk, preferred_element_type=jnp.float32)
                     + running)
        running = running + jnp.sum(blk, axis=0, keepdims=True)
    rank = jnp.concatenate(ranks, axis=0)
    counts = running
    padded = jnp.floor((counts + (TM - 1)) / TM) * TM
    triu8 = (lax.broadcasted_iota(jnp.int32, (E, E), 0)
             < lax.broadcasted_iota(jnp.int32, (E, E), 1)).astype(jnp.float32)
    offsets = jnp.dot(padded, triu8, preferred_element_type=jnp.float32)
    dest_f = jnp.sum(onehot * (offsets + rank - 1.0), axis=1, keepdims=True)
    dest_ref[...] = dest_f.astype(jnp.int32)

    rb = (lax.broadcasted_iota(jnp.int32, (NB, E), 0) * TM).astype(jnp.float32)
    be_ref[...] = (jnp.sum((rb >= offsets).astype(jnp.float32),
                           axis=1, keepdims=True) - 1.0).astype(jnp.int32)


def _prep(sel_col):
    return pl.pallas_call(
        _prep_body,
        in_specs=[pl.BlockSpec((P, 1), lambda: (0, 0))],
        out_specs=[
            pl.BlockSpec((P, 1), lambda: (0, 0)),
            pl.BlockSpec((NB, 1), lambda: (0, 0)),
        ],
        out_shape=[
            jax.ShapeDtypeStruct((P, 1), jnp.int32),
            jax.ShapeDtypeStruct((NB, 1), jnp.int32),
        ],
    )(sel_col)


def _sc_dispatch(x, idx_a, idx_b):
    info = plsc.get_sparse_core_info()
    nw = info.num_cores * info.num_subcores
    t_per_w = T // nw
    mesh = plsc.VectorSubcoreMesh(core_axis_name="c", subcore_axis_name="s")

    @functools.partial(
        pl.kernel, mesh=mesh,
        out_type=jax.ShapeDtypeStruct((PPAD, D), jnp.float32),
        scratch_types=[
            pltpu.VMEM((t_per_w,), jnp.int32),
            pltpu.VMEM((t_per_w,), jnp.int32),
            pltpu.VMEM((t_per_w, D), jnp.float32),
            pltpu.SemaphoreType.DMA,
            pltpu.SemaphoreType.DMA,
        ],
    )
    def k(x_hbm, ia_hbm, ib_hbm, out_hbm, ia_v, ib_v, rows_v, sem_a, sem_b):
        wid = lax.axis_index("s") * info.num_cores + lax.axis_index("c")
        base = wid * t_per_w
        pltpu.sync_copy(ia_hbm.at[pl.ds(base, t_per_w)], ia_v)
        pltpu.sync_copy(ib_hbm.at[pl.ds(base, t_per_w)], ib_v)
        pltpu.sync_copy(x_hbm.at[pl.ds(base, t_per_w)], rows_v)
        ca = pltpu.async_copy(rows_v, out_hbm.at[ia_v], sem_a)
        cb = pltpu.async_copy(rows_v, out_hbm.at[ib_v], sem_b)
        ca.wait()
        cb.wait()

    return k(x, idx_a, idx_b)


def _sc_collect(table, idx):
    info = plsc.get_sparse_core_info()
    nw = info.num_cores * info.num_subcores
    b_per_w = P // nw
    chunk = 64
    mesh = plsc.VectorSubcoreMesh(core_axis_name="c", subcore_axis_name="s")

    @functools.partial(
        pl.kernel, mesh=mesh,
        out_type=jax.ShapeDtypeStruct((P, D), jnp.float32),
        scratch_types=[
            pltpu.VMEM((chunk,), jnp.int32),
            pltpu.VMEM((chunk, D), jnp.float32),
            pltpu.SemaphoreType.DMA,
        ],
    )
    def k(table_hbm, idx_hbm, out_hbm, idx_v, rows_v, sem):
        wid = lax.axis_index("s") * info.num_cores + lax.axis_index("c")
        base = wid * b_per_w
        for c in range(b_per_w // chunk):
            off = base + c * chunk
            pltpu.sync_copy(idx_hbm.at[pl.ds(off, chunk)], idx_v)
            pltpu.async_copy(table_hbm.at[idx_v], rows_v, sem).wait()
            pltpu.sync_copy(rows_v, out_hbm.at[pl.ds(off, chunk)])

    return k(table, idx)


def _stage_a_body(be_ref, xs_ref, w1_ref, w3_ref, h_ref):
    xs = xs_ref[...]
    a = jnp.dot(xs, w1_ref[0], preferred_element_type=jnp.float32,
                precision=lax.Precision.DEFAULT)
    b = jnp.dot(xs, w3_ref[0], preferred_element_type=jnp.float32,
                precision=lax.Precision.DEFAULT)
    h_ref[...] = a * jax.nn.sigmoid(a) * b


def _stage_a(xs, w1, w3, be):
    grid = (F // FN, NB)
    return pl.pallas_call(
        _stage_a_body,
        grid_spec=pltpu.PrefetchScalarGridSpec(
            num_scalar_prefetch=1,
            grid=grid,
            in_specs=[
                pl.BlockSpec((TM, D), lambda fb, rb, be: (rb, 0)),
                pl.BlockSpec((1, D, FN), lambda fb, rb, be: (be[rb], 0, fb)),
                pl.BlockSpec((1, D, FN), lambda fb, rb, be: (be[rb], 0, fb)),
            ],
            out_specs=pl.BlockSpec((TM, FN), lambda fb, rb, be: (rb, fb)),
        ),
        out_shape=jax.ShapeDtypeStruct((PPAD, F), jnp.float32),
    )(be, xs, w1, w3)


def _stage_b_body(be_ref, h_ref, w2_ref, op_ref):
    op_ref[...] = jnp.dot(h_ref[...], w2_ref[0],
                          preferred_element_type=jnp.float32,
                          precision=lax.Precision.DEFAULT)


def _stage_b(h, w2, be):
    grid = (D // DN, NB)
    return pl.pallas_call(
        _stage_b_body,
        grid_spec=pltpu.PrefetchScalarGridSpec(
            num_scalar_prefetch=1,
            grid=grid,
            in_specs=[
                pl.BlockSpec((TM, F), lambda db, rb, be: (rb, 0)),
                pl.BlockSpec((1, F, DN), lambda db, rb, be: (be[rb], 0, db)),
            ],
            out_specs=pl.BlockSpec((TM, DN), lambda db, rb, be: (rb, db)),
        ),
        out_shape=jax.ShapeDtypeStruct((PPAD, D), jnp.float32),
    )(be, h, w2)


def _combine_body(g_ref, rw_ref, out_ref):
    g = g_ref[...]
    rw = rw_ref[...]
    out_ref[...] = (g[:, :D] * rw[:, 0:1] + g[:, D:] * rw[:, 1:2])[None]


def _combine(g2, rw):
    bt = 256
    return pl.pallas_call(
        _combine_body,
        grid=(T // bt,),
        in_specs=[
            pl.BlockSpec((bt, K * D), lambda i: (i, 0)),
            pl.BlockSpec((bt, K), lambda i: (i, 0)),
        ],
        out_specs=pl.BlockSpec((1, bt, D), lambda i: (0, i, 0)),
        out_shape=jax.ShapeDtypeStruct((1, T, D), jnp.float32),
    )(g2, rw)


def kernel(hidden_states, gate_w, w1, w2, w3):
    x = hidden_states.reshape(T, D)
    router_logits, rw, sel = _router(x, gate_w)

    dest, be2 = _prep(sel.reshape(P, 1))
    be = be2.reshape(NB)
    dest2 = dest.reshape(T, K)
    idx_a = dest2[:, 0]
    idx_b = dest2[:, 1]
    pos = dest.reshape(P)

    xs = _sc_dispatch(x, idx_a, idx_b)
    h = _stage_a(xs, w1, w3, be)
    op = _stage_b(h, w2, be)
    g = _sc_collect(op, pos)
    out = _combine(g.reshape(T, K * D), rw)
    return out, router_logits


# speedup vs baseline: 1.5554x; 1.0598x over previous
import functools

import jax
import jax.numpy as jnp
from jax import lax
from jax.experimental import pallas as pl
from jax.experimental.pallas import tpu as pltpu
from jax.experimental.pallas import tpu_sc as plsc

T = 2048
D = 1024
F = 3584
E = 8
K = 2
P = T * K
TM = 256
PPAD = P + E * TM
NB = PPAD // TM
FN = 1792
DN = 1024


def _router_prep_body(x_ref, gw_ref, logits_ref, rw_ref, ia_ref, ib_ref,
                      pos_ref, be_ref):
    x = x_ref[...]
    logits = jnp.dot(x, gw_ref[...], preferred_element_type=jnp.float32)
    logits_ref[...] = logits
    iota = lax.broadcasted_iota(jnp.int32, logits.shape, 1)
    m1 = jnp.max(logits, axis=1, keepdims=True)
    a1 = jnp.min(jnp.where(logits == m1, iota, E), axis=1, keepdims=True)
    rest = jnp.where(iota == a1, -jnp.inf, logits)
    m2 = jnp.max(rest, axis=1, keepdims=True)
    a2 = jnp.min(jnp.where(rest == m2, iota, E), axis=1, keepdims=True)
    e2 = jnp.exp(m2 - m1)
    denom = 1.0 + e2
    rw_ref[...] = jnp.concatenate([1.0 / denom, e2 / denom], axis=1)

    sel_col = jnp.concatenate([a1, a2], axis=0)
    onehot = (sel_col == lax.broadcasted_iota(jnp.int32, (P, E), 1)
              ).astype(jnp.float32)
    ch = 512
    tri = (lax.broadcasted_iota(jnp.int32, (ch, ch), 0)
           >= lax.broadcasted_iota(jnp.int32, (ch, ch), 1)).astype(jnp.float32)
    running = jnp.zeros((1, E), jnp.float32)
    ranks = []
    for i in range(P // ch):
        blk = onehot[i * ch:(i + 1) * ch, :]
        ranks.append(jnp.dot(tri, blk, preferred_element_type=jnp.float32)
                     + running)
        running = running + jnp.sum(blk, axis=0, keepdims=True)
    rank = jnp.concatenate(ranks, axis=0)
    counts = running
    padded = jnp.floor((counts + (TM - 1)) / TM) * TM
    triu8 = (lax.broadcasted_iota(jnp.int32, (E, E), 0)
             < lax.broadcasted_iota(jnp.int32, (E, E), 1)).astype(jnp.float32)
    offsets = jnp.dot(padded, triu8, preferred_element_type=jnp.float32)
    dest_f = jnp.sum(onehot * (offsets + rank - 1.0), axis=1, keepdims=True)
    dest = dest_f.astype(jnp.int32)
    ia_ref[...] = dest[:T]
    ib_ref[...] = dest[T:]
    pos_ref[...] = dest

    rb = (lax.broadcasted_iota(jnp.int32, (NB, E), 0) * TM).astype(jnp.float32)
    be_ref[...] = (jnp.sum((rb >= offsets).astype(jnp.float32),
                           axis=1, keepdims=True) - 1.0).astype(jnp.int32)


def _router_prep(x, gate_w):
    return pl.pallas_call(
        _router_prep_body,
        in_specs=[
            pl.BlockSpec((T, D), lambda: (0, 0)),
            pl.BlockSpec((D, E), lambda: (0, 0)),
        ],
        out_specs=[
            pl.BlockSpec((T, E), lambda: (0, 0)),
            pl.BlockSpec((T, K), lambda: (0, 0)),
            pl.BlockSpec((T, 1), lambda: (0, 0)),
            pl.BlockSpec((T, 1), lambda: (0, 0)),
            pl.BlockSpec((P, 1), lambda: (0, 0)),
            pl.BlockSpec((NB, 1), lambda: (0, 0)),
        ],
        out_shape=[
            jax.ShapeDtypeStruct((T, E), jnp.float32),
            jax.ShapeDtypeStruct((T, K), jnp.float32),
            jax.ShapeDtypeStruct((T, 1), jnp.int32),
            jax.ShapeDtypeStruct((T, 1), jnp.int32),
            jax.ShapeDtypeStruct((P, 1), jnp.int32),
            jax.ShapeDtypeStruct((NB, 1), jnp.int32),
        ],
    )(x, gate_w)


def _sc_dispatch(x, idx_a, idx_b):
    info = plsc.get_sparse_core_info()
    nw = info.num_cores * info.num_subcores
    t_per_w = T // nw
    mesh = plsc.VectorSubcoreMesh(core_axis_name="c", subcore_axis_name="s")

    @functools.partial(
        pl.kernel, mesh=mesh,
        out_type=jax.ShapeDtypeStruct((PPAD, D), jnp.float32),
        scratch_types=[
            pltpu.VMEM((t_per_w,), jnp.int32),
            pltpu.VMEM((t_per_w,), jnp.int32),
            pltpu.VMEM((t_per_w, D), jnp.float32),
            pltpu.SemaphoreType.DMA,
            pltpu.SemaphoreType.DMA,
        ],
    )
    def k(x_hbm, ia_hbm, ib_hbm, out_hbm, ia_v, ib_v, rows_v, sem_a, sem_b):
        wid = lax.axis_index("s") * info.num_cores + lax.axis_index("c")
        base = wid * t_per_w
        pltpu.sync_copy(ia_hbm.at[pl.ds(base, t_per_w)], ia_v)
        pltpu.sync_copy(ib_hbm.at[pl.ds(base, t_per_w)], ib_v)
        pltpu.sync_copy(x_hbm.at[pl.ds(base, t_per_w)], rows_v)
        ca = pltpu.async_copy(rows_v, out_hbm.at[ia_v], sem_a)
        cb = pltpu.async_copy(rows_v, out_hbm.at[ib_v], sem_b)
        ca.wait()
        cb.wait()

    return k(x, idx_a, idx_b)


def _sc_collect(table, idx):
    info = plsc.get_sparse_core_info()
    nw = info.num_cores * info.num_subcores
    b_per_w = P // nw
    chunk = 64
    mesh = plsc.VectorSubcoreMesh(core_axis_name="c", subcore_axis_name="s")

    @functools.partial(
        pl.kernel, mesh=mesh,
        out_type=jax.ShapeDtypeStruct((P, D), jnp.float32),
        scratch_types=[
            pltpu.VMEM((chunk,), jnp.int32),
            pltpu.VMEM((chunk, D), jnp.float32),
            pltpu.SemaphoreType.DMA,
        ],
    )
    def k(table_hbm, idx_hbm, out_hbm, idx_v, rows_v, sem):
        wid = lax.axis_index("s") * info.num_cores + lax.axis_index("c")
        base = wid * b_per_w
        for c in range(b_per_w // chunk):
            off = base + c * chunk
            pltpu.sync_copy(idx_hbm.at[pl.ds(off, chunk)], idx_v)
            pltpu.async_copy(table_hbm.at[idx_v], rows_v, sem).wait()
            pltpu.sync_copy(rows_v, out_hbm.at[pl.ds(off, chunk)])

    return k(table, idx)


def _stage_a_body(be_ref, xs_ref, w1_ref, w3_ref, h_ref):
    xs = xs_ref[...]
    a = jnp.dot(xs, w1_ref[0], preferred_element_type=jnp.float32,
                precision=lax.Precision.DEFAULT)
    b = jnp.dot(xs, w3_ref[0], preferred_element_type=jnp.float32,
                precision=lax.Precision.DEFAULT)
    h_ref[...] = a * jax.nn.sigmoid(a) * b


def _stage_a(xs, w1, w3, be):
    grid = (F // FN, NB)
    return pl.pallas_call(
        _stage_a_body,
        grid_spec=pltpu.PrefetchScalarGridSpec(
            num_scalar_prefetch=1,
            grid=grid,
            in_specs=[
                pl.BlockSpec((TM, D), lambda fb, rb, be: (rb, 0)),
                pl.BlockSpec((1, D, FN), lambda fb, rb, be: (be[rb], 0, fb)),
                pl.BlockSpec((1, D, FN), lambda fb, rb, be: (be[rb], 0, fb)),
            ],
            out_specs=pl.BlockSpec((TM, FN), lambda fb, rb, be: (rb, fb)),
        ),
        out_shape=jax.ShapeDtypeStruct((PPAD, F), jnp.float32),
    )(be, xs, w1, w3)


def _stage_b_body(be_ref, h_ref, w2_ref, op_ref):
    op_ref[...] = jnp.dot(h_ref[...], w2_ref[0],
                          preferred_element_type=jnp.float32,
                          precision=lax.Precision.DEFAULT)


def _stage_b(h, w2, be):
    grid = (D // DN, NB)
    return pl.pallas_call(
        _stage_b_body,
        grid_spec=pltpu.PrefetchScalarGridSpec(
            num_scalar_prefetch=1,
            grid=grid,
            in_specs=[
                pl.BlockSpec((TM, F), lambda db, rb, be: (rb, 0)),
                pl.BlockSpec((1, F, DN), lambda db, rb, be: (be[rb], 0, db)),
            ],
            out_specs=pl.BlockSpec((TM, DN), lambda db, rb, be: (rb, db)),
        ),
        out_shape=jax.ShapeDtypeStruct((PPAD, D), jnp.float32),
    )(be, h, w2)


def _combine_body(g0_ref, g1_ref, rw_ref, out_ref):
    rw = rw_ref[...]
    out_ref[...] = (g0_ref[...] * rw[:, 0:1] + g1_ref[...] * rw[:, 1:2])[None]


def _combine(g, rw):
    bt = 256
    return pl.pallas_call(
        _combine_body,
        grid=(T // bt,),
        in_specs=[
            pl.BlockSpec((bt, D), lambda i: (i, 0)),
            pl.BlockSpec((bt, D), lambda i: (i + T // bt, 0)),
            pl.BlockSpec((bt, K), lambda i: (i, 0)),
        ],
        out_specs=pl.BlockSpec((1, bt, D), lambda i: (0, i, 0)),
        out_shape=jax.ShapeDtypeStruct((1, T, D), jnp.float32),
    )(g, g, rw)


def kernel(hidden_states, gate_w, w1, w2, w3):
    x = hidden_states.reshape(T, D)
    router_logits, rw, ia, ib, pos, be2 = _router_prep(x, gate_w)
    be = be2.reshape(NB)

    xs = _sc_dispatch(x, ia.reshape(T), ib.reshape(T))
    h = _stage_a(xs, w1, w3, be)
    op = _stage_b(h, w2, be)
    g = _sc_collect(op, pos.reshape(P))
    out = _combine(g, rw)
    return out, router_logits


# speedup vs baseline: 1.6167x; 1.0394x over previous
import functools

import jax
import jax.numpy as jnp
from jax import lax
from jax.experimental import pallas as pl
from jax.experimental.pallas import tpu as pltpu
from jax.experimental.pallas import tpu_sc as plsc

T = 2048
D = 1024
F = 3584
E = 8
K = 2
P = T * K
TM = 256
PPAD = P + E * TM
NB = PPAD // TM
FN = 1792
DN = 1024


def _router_prep_body(x_ref, gw_ref, logits_ref, rw_ref, ia_ref, ib_ref,
                      pos_ref, be_ref):
    x = x_ref[...]
    logits = jnp.dot(x, gw_ref[...], preferred_element_type=jnp.float32)
    logits_ref[...] = logits
    iota = lax.broadcasted_iota(jnp.int32, logits.shape, 1)
    m1 = jnp.max(logits, axis=1, keepdims=True)
    a1 = jnp.min(jnp.where(logits == m1, iota, E), axis=1, keepdims=True)
    rest = jnp.where(iota == a1, -jnp.inf, logits)
    m2 = jnp.max(rest, axis=1, keepdims=True)
    a2 = jnp.min(jnp.where(rest == m2, iota, E), axis=1, keepdims=True)
    e2 = jnp.exp(m2 - m1)
    denom = 1.0 + e2
    rw_ref[...] = jnp.concatenate([1.0 / denom, e2 / denom], axis=1)

    sel_col = jnp.concatenate([a1, a2], axis=0)
    onehot = (sel_col == lax.broadcasted_iota(jnp.int32, (P, E), 1)
              ).astype(jnp.float32)
    ch = 512
    tri = (lax.broadcasted_iota(jnp.int32, (ch, ch), 0)
           >= lax.broadcasted_iota(jnp.int32, (ch, ch), 1)).astype(jnp.float32)
    running = jnp.zeros((1, E), jnp.float32)
    ranks = []
    for i in range(P // ch):
        blk = onehot[i * ch:(i + 1) * ch, :]
        ranks.append(jnp.dot(tri, blk, preferred_element_type=jnp.float32)
                     + running)
        running = running + jnp.sum(blk, axis=0, keepdims=True)
    rank = jnp.concatenate(ranks, axis=0)
    counts = running
    padded = jnp.floor((counts + (TM - 1)) / TM) * TM
    triu8 = (lax.broadcasted_iota(jnp.int32, (E, E), 0)
             < lax.broadcasted_iota(jnp.int32, (E, E), 1)).astype(jnp.float32)
    offsets = jnp.dot(padded, triu8, preferred_element_type=jnp.float32)
    dest_f = jnp.sum(onehot * (offsets + rank - 1.0), axis=1, keepdims=True)
    dest = dest_f.astype(jnp.int32)
    ia_ref[...] = dest[:T]
    ib_ref[...] = dest[T:]
    pos_ref[...] = dest

    rb = (lax.broadcasted_iota(jnp.int32, (NB, E), 0) * TM).astype(jnp.float32)
    be_ref[...] = (jnp.sum((rb >= offsets).astype(jnp.float32),
                           axis=1, keepdims=True) - 1.0).astype(jnp.int32)


def _router_prep(x, gate_w):
    return pl.pallas_call(
        _router_prep_body,
        in_specs=[
            pl.BlockSpec((T, D), lambda: (0, 0)),
            pl.BlockSpec((D, E), lambda: (0, 0)),
        ],
        out_specs=[
            pl.BlockSpec((T, E), lambda: (0, 0)),
            pl.BlockSpec((T, K), lambda: (0, 0)),
            pl.BlockSpec((T, 1), lambda: (0, 0)),
            pl.BlockSpec((T, 1), lambda: (0, 0)),
            pl.BlockSpec((P, 1), lambda: (0, 0)),
            pl.BlockSpec((NB, 1), lambda: (0, 0)),
        ],
        out_shape=[
            jax.ShapeDtypeStruct((T, E), jnp.float32),
            jax.ShapeDtypeStruct((T, K), jnp.float32),
            jax.ShapeDtypeStruct((T, 1), jnp.int32),
            jax.ShapeDtypeStruct((T, 1), jnp.int32),
            jax.ShapeDtypeStruct((P, 1), jnp.int32),
            jax.ShapeDtypeStruct((NB, 1), jnp.int32),
        ],
    )(x, gate_w)


def _sc_dispatch(x, idx_a, idx_b):
    info = plsc.get_sparse_core_info()
    nw = info.num_cores * info.num_subcores
    t_per_w = T // nw
    mesh = plsc.VectorSubcoreMesh(core_axis_name="c", subcore_axis_name="s")

    @functools.partial(
        pl.kernel, mesh=mesh,
        out_type=jax.ShapeDtypeStruct((PPAD, D), jnp.float32),
        scratch_types=[
            pltpu.VMEM((t_per_w,), jnp.int32),
            pltpu.VMEM((t_per_w,), jnp.int32),
            pltpu.VMEM((t_per_w, D), jnp.float32),
            pltpu.SemaphoreType.DMA,
            pltpu.SemaphoreType.DMA,
        ],
    )
    def k(x_hbm, ia_hbm, ib_hbm, out_hbm, ia_v, ib_v, rows_v, sem_a, sem_b):
        wid = lax.axis_index("s") * info.num_cores + lax.axis_index("c")
        base = wid * t_per_w
        pltpu.sync_copy(ia_hbm.at[pl.ds(base, t_per_w)], ia_v)
        pltpu.sync_copy(ib_hbm.at[pl.ds(base, t_per_w)], ib_v)
        pltpu.sync_copy(x_hbm.at[pl.ds(base, t_per_w)], rows_v)
        ca = pltpu.async_copy(rows_v, out_hbm.at[ia_v], sem_a)
        cb = pltpu.async_copy(rows_v, out_hbm.at[ib_v], sem_b)
        ca.wait()
        cb.wait()

    return k(x, idx_a, idx_b)


def _sc_collect(table, idx):
    info = plsc.get_sparse_core_info()
    nw = info.num_cores * info.num_subcores
    b_per_w = P // nw
    chunk = 64
    mesh = plsc.VectorSubcoreMesh(core_axis_name="c", subcore_axis_name="s")

    @functools.partial(
        pl.kernel, mesh=mesh,
        out_type=jax.ShapeDtypeStruct((P, D), jnp.float32),
        scratch_types=[
            pltpu.VMEM((chunk,), jnp.int32),
            pltpu.VMEM((chunk, D), jnp.float32),
            pltpu.SemaphoreType.DMA,
        ],
    )
    def k(table_hbm, idx_hbm, out_hbm, idx_v, rows_v, sem):
        wid = lax.axis_index("s") * info.num_cores + lax.axis_index("c")
        base = wid * b_per_w
        for c in range(b_per_w // chunk):
            off = base + c * chunk
            pltpu.sync_copy(idx_hbm.at[pl.ds(off, chunk)], idx_v)
            pltpu.async_copy(table_hbm.at[idx_v], rows_v, sem).wait()
            pltpu.sync_copy(rows_v, out_hbm.at[pl.ds(off, chunk)])

    return k(table, idx)


def _stage_a_body(be_ref, xs_ref, w1_ref, w3_ref, h_ref):
    xs = xs_ref[...]
    a = jnp.dot(xs, w1_ref[0], preferred_element_type=jnp.float32,
                precision=lax.Precision.DEFAULT)
    b = jnp.dot(xs, w3_ref[0], preferred_element_type=jnp.float32,
                precision=lax.Precision.DEFAULT)
    h_ref[...] = (a * jax.nn.sigmoid(a) * b).astype(jnp.bfloat16)


def _stage_a(xs, w1, w3, be):
    grid = (F // FN, NB)
    return pl.pallas_call(
        _stage_a_body,
        grid_spec=pltpu.PrefetchScalarGridSpec(
            num_scalar_prefetch=1,
            grid=grid,
            in_specs=[
                pl.BlockSpec((TM, D), lambda fb, rb, be: (rb, 0)),
                pl.BlockSpec((1, D, FN), lambda fb, rb, be: (be[rb], 0, fb)),
                pl.BlockSpec((1, D, FN), lambda fb, rb, be: (be[rb], 0, fb)),
            ],
            out_specs=pl.BlockSpec((TM, FN), lambda fb, rb, be: (rb, fb)),
        ),
        out_shape=jax.ShapeDtypeStruct((PPAD, F), jnp.bfloat16),
    )(be, xs, w1, w3)


def _stage_b_body(be_ref, h_ref, w2_ref, op_ref):
    op_ref[...] = jnp.dot(h_ref[...].astype(jnp.float32), w2_ref[0],
                          preferred_element_type=jnp.float32,
                          precision=lax.Precision.DEFAULT)


def _stage_b(h, w2, be):
    grid = (D // DN, NB)
    return pl.pallas_call(
        _stage_b_body,
        grid_spec=pltpu.PrefetchScalarGridSpec(
            num_scalar_prefetch=1,
            grid=grid,
            in_specs=[
                pl.BlockSpec((TM, F), lambda db, rb, be: (rb, 0)),
                pl.BlockSpec((1, F, DN), lambda db, rb, be: (be[rb], 0, db)),
            ],
            out_specs=pl.BlockSpec((TM, DN), lambda db, rb, be: (rb, db)),
        ),
        out_shape=jax.ShapeDtypeStruct((PPAD, D), jnp.float32),
    )(be, h, w2)


def _combine_body(g0_ref, g1_ref, rw_ref, out_ref):
    rw = rw_ref[...]
    out_ref[...] = (g0_ref[...] * rw[:, 0:1] + g1_ref[...] * rw[:, 1:2])[None]


def _combine(g, rw):
    bt = 256
    return pl.pallas_call(
        _combine_body,
        grid=(T // bt,),
        in_specs=[
            pl.BlockSpec((bt, D), lambda i: (i, 0)),
            pl.BlockSpec((bt, D), lambda i: (i + T // bt, 0)),
            pl.BlockSpec((bt, K), lambda i: (i, 0)),
        ],
        out_specs=pl.BlockSpec((1, bt, D), lambda i: (0, i, 0)),
        out_shape=jax.ShapeDtypeStruct((1, T, D), jnp.float32),
    )(g, g, rw)


def kernel(hidden_states, gate_w, w1, w2, w3):
    x = hidden_states.reshape(T, D)
    router_logits, rw, ia, ib, pos, be2 = _router_prep(x, gate_w)
    be = be2.reshape(NB)

    xs = _sc_dispatch(x, ia.reshape(T), ib.reshape(T))
    h = _stage_a(xs, w1, w3, be)
    op = _stage_b(h, w2, be)
    g = _sc_collect(op, pos.reshape(P))
    out = _combine(g, rw)
    return out, router_logits
